# Initial kernel scaffold; baseline (speedup 1.0000x reference)
import jax
import jax.numpy as jnp
from jax.experimental import pallas as pl


def kernel(x_user, x_item, edge_attr, Wl0_ui, bl0_ui, Wr0_ui, Wl0_iu, bl0_iu, Wr0_iu, Wl1_ui, bl1_ui, Wr1_ui, Wl1_iu, bl1_iu, Wr1_iu, W1, b1, W2, b2, W3, b3, ei_ui, ei_iu, edge_label_index):
    raise NotImplementedError("write your pallas kernel here")


# speedup vs baseline: 3.3542x; 3.3542x over previous
import functools

import jax
import jax.numpy as jnp
from jax import lax
from jax.experimental import pallas as pl
from jax.experimental.pallas import tpu as pltpu
from jax.experimental.pallas import tpu_sc as plsc

N = 10000
ND = 2 * N
E = 320000
E_ALL = 2 * E
D = 128
EL = 100000
DE = 16

NC = 2
NS = 16
NW = NC * NS

CH = 80
EPT = E // NS
NCHUNK = EPT // CH
RPT = 640
NPAD = NS * RPT

GPAD = 204800
GPT = GPAD // NW
GCH = 80
GNCH = GPT // GCH

_f32 = jnp.float32


@functools.cache
def _build_sc_agg():
    mesh = plsc.VectorSubcoreMesh(core_axis_name="c", subcore_axis_name="s")

    @functools.partial(
        pl.kernel,
        mesh=mesh,
        out_type=[
            jax.ShapeDtypeStruct((ND, D), _f32),
            jax.ShapeDtypeStruct((ND,), _f32),
        ],
        scratch_types=[
            pltpu.VMEM((CH,), jnp.int32),
            pltpu.VMEM((CH,), jnp.int32),
            pltpu.VMEM((CH, D), _f32),
            pltpu.VMEM((CH,), _f32),
            pltpu.VMEM((CH, D), _f32),
            pltpu.VMEM((CH,), _f32),
            pltpu.VMEM((RPT,), _f32),
            pltpu.VMEM_SHARED((NPAD, D), _f32),
            pltpu.VMEM_SHARED((NPAD,), _f32),
            pltpu.SemaphoreType.DMA,
        ],
    )
    def sc_agg(table, srcs, dsts, agg_out, cnt_out,
               sidx, didx, rows, onev, zb, zc, cvec, sagg, scnt, sem):
        c = lax.axis_index("c")
        s = lax.axis_index("s")

        def fill_vecs(j, carry):
            onev[pl.ds(j * 16, 16)] = jnp.ones((16,), _f32)
            zc[pl.ds(j * 16, 16)] = jnp.zeros((16,), _f32)
            return carry

        lax.fori_loop(0, CH // 16, fill_vecs, 0)

        def fill_zero(j, carry):
            for g in range(D // 16):
                zb[j, pl.ds(g * 16, 16)] = jnp.zeros((16,), _f32)
            return carry

        lax.fori_loop(0, CH, fill_zero, 0)

        for r in range(RPT // CH):
            pltpu.sync_copy(zb, sagg.at[pl.ds(s * RPT + r * CH, CH)])
            pltpu.sync_copy(zc, scnt.at[pl.ds(s * RPT + r * CH, CH)])
        plsc.subcore_barrier()

        base = c * E + s * EPT

        def body(i, carry):
            eb = base + i * CH
            pltpu.sync_copy(srcs.at[pl.ds(eb, CH)], sidx)
            pltpu.sync_copy(dsts.at[pl.ds(eb, CH)], didx)
            pltpu.async_copy(table.at[sidx], rows, sem).wait()
            pltpu.sync_copy(rows, sagg.at[didx], add=True)
            pltpu.sync_copy(onev, scnt.at[didx], add=True)
            return carry

        lax.fori_loop(0, NCHUNK, body, 0)
        plsc.subcore_barrier()
        ob = c * N + s * RPT
        last = N - (NS - 1) * RPT

        pltpu.sync_copy(scnt.at[pl.ds(s * RPT, RPT)], cvec)

        @pl.when(s < NS - 1)
        def _():
            pltpu.sync_copy(sagg.at[pl.ds(s * RPT, RPT)],
                            agg_out.at[pl.ds(ob, RPT)])
            pltpu.sync_copy(cvec, cnt_out.at[pl.ds(ob, RPT)])

        @pl.when(s == NS - 1)
        def _():
            pltpu.sync_copy(sagg.at[pl.ds((NS - 1) * RPT, last)],
                            agg_out.at[pl.ds(c * N + (NS - 1) * RPT, last)])
            pltpu.sync_copy(cvec.at[pl.ds(0, last)],
                            cnt_out.at[pl.ds(c * N + (NS - 1) * RPT, last)])

    return sc_agg


def _sc_agg(*args):
    return _build_sc_agg()(*args)


@functools.cache
def _build_sc_gather():
    mesh = plsc.VectorSubcoreMesh(core_axis_name="c", subcore_axis_name="s")

    @functools.partial(
        pl.kernel,
        mesh=mesh,
        out_type=jax.ShapeDtypeStruct((GPAD, D), _f32),
        scratch_types=[
            pltpu.VMEM((GCH,), jnp.int32),
            pltpu.VMEM((GCH, D), _f32),
            pltpu.SemaphoreType.DMA,
        ],
    )
    def sc_gather(z, idxs, out, vidx, rows, sem):
        c = lax.axis_index("c")
        s = lax.axis_index("s")
        base = (s * NC + c) * GPT

        def body(i, carry):
            gb = base + i * GCH
            pltpu.sync_copy(idxs.at[pl.ds(gb, GCH)], vidx)
            pltpu.async_copy(z.at[vidx], rows, sem).wait()
            pltpu.sync_copy(rows, out.at[pl.ds(gb, GCH)])
            return carry

        lax.fori_loop(0, GNCH, body, 0)

    return sc_gather


def _sc_gather(*args):
    return _build_sc_gather()(*args)


BR = 400
NB = ND // BR
HB = NB // 2

def _transform_body(x_ref, w_ref, o_ref):
    o_ref[...] = jnp.dot(x_ref[...], w_ref[0], preferred_element_type=_f32)


def _transform(xd, wstack):
    return pl.pallas_call(
        _transform_body,
        grid=(NB,),
        in_specs=[
            pl.BlockSpec((BR, D), lambda i: ((i + HB) % NB, 0)),
            pl.BlockSpec((1, D, D), lambda i: (i // HB, 0, 0)),
        ],
        out_specs=pl.BlockSpec((BR, D), lambda i: (i, 0)),
        out_shape=jax.ShapeDtypeStruct((ND, D), _f32),
    )(xd, wstack)


def _epilogue_body(relu, agg_ref, cnt_ref, x_ref, w_ref, b_ref, o_ref):
    cnt = jnp.maximum(cnt_ref[...], 1.0)
    h = agg_ref[...] / cnt + jnp.dot(
        x_ref[...], w_ref[0], preferred_element_type=_f32) + b_ref[0, 0]
    if relu:
        h = jnp.maximum(h, 0.0)
    o_ref[...] = h


def _epilogue(agg, cnt, xd, wstack, bstack, relu):
    return pl.pallas_call(
        functools.partial(_epilogue_body, relu),
        grid=(NB,),
        in_specs=[
            pl.BlockSpec((BR, D), lambda i: (i, 0)),
            pl.BlockSpec((BR, 1), lambda i: (i, 0)),
            pl.BlockSpec((BR, D), lambda i: (i, 0)),
            pl.BlockSpec((1, D, D), lambda i: (i // HB, 0, 0)),
            pl.BlockSpec((1, 1, D), lambda i: (i // HB, 0, 0)),
        ],
        out_specs=pl.BlockSpec((BR, D), lambda i: (i, 0)),
        out_shape=jax.ShapeDtypeStruct((ND, D), _f32),
    )(agg, cnt[:, None], xd, wstack, bstack[:, None])


MR = 400
MB = EL // MR


def _mlp_body(gu_ref, gi_ref, ea_ref, w1u_ref, w1i_ref, w1e_ref, b1_ref,
              w2_ref, b2_ref, w3_ref, b3_ref, o_ref):
    z = (jnp.dot(gu_ref[...], w1u_ref[...], preferred_element_type=_f32)
         + jnp.dot(gi_ref[...], w1i_ref[...], preferred_element_type=_f32)
         + jnp.dot(ea_ref[...], w1e_ref[...], preferred_element_type=_f32)
         + b1_ref[...])
    z = jnp.maximum(z, 0.0)
    z = jnp.maximum(jnp.dot(z, w2_ref[...], preferred_element_type=_f32)
                    + b2_ref[...], 0.0)
    o_ref[...] = jnp.dot(z, w3_ref[...], preferred_element_type=_f32) + b3_ref[...]


def _mlp(g, ea, w1u, w1i, w1e, b1, w2, b2, w3, b3):
    full = lambda i: (0, 0)
    return pl.pallas_call(
        _mlp_body,
        grid=(MB,),
        in_specs=[
            pl.BlockSpec((MR, D), lambda i: (i, 0)),
            pl.BlockSpec((MR, D), lambda i: (i + MB, 0)),
            pl.BlockSpec((MR, DE), lambda i: (i, 0)),
            pl.BlockSpec((D, D), full),
            pl.BlockSpec((D, D), full),
            pl.BlockSpec((DE, D), full),
            pl.BlockSpec((1, D), full),
            pl.BlockSpec((D, D), full),
            pl.BlockSpec((1, D), full),
            pl.BlockSpec((D, 2), full),
            pl.BlockSpec((1, 2), full),
        ],
        out_specs=pl.BlockSpec((MR, 2), lambda i: (i, 0)),
        out_shape=jax.ShapeDtypeStruct((EL, 2), _f32),
    )(g, g, ea, w1u, w1i, w1e, b1, w2, b2, w3, b3)


def kernel(x_user, x_item, edge_attr, Wl0_ui, bl0_ui, Wr0_ui, Wl0_iu, bl0_iu,
           Wr0_iu, Wl1_ui, bl1_ui, Wr1_ui, Wl1_iu, bl1_iu, Wr1_iu, W1, b1,
           W2, b2, W3, b3, ei_ui, ei_iu, edge_label_index):
    xd = jnp.concatenate([x_item, x_user], axis=0)
    srcs = jnp.concatenate([ei_ui[0], ei_iu[0] + N])
    dsts = jnp.concatenate([ei_ui[1], ei_iu[1]])

    t0 = _transform(xd, jnp.stack([Wl0_ui, Wl0_iu]))
    agg0, cnt = _sc_agg(t0, srcs, dsts)
    hd = _epilogue(agg0, cnt, xd,
                   jnp.stack([Wr0_ui, Wr0_iu]),
                   jnp.stack([bl0_ui, bl0_iu]), relu=True)
    t1 = _transform(hd, jnp.stack([Wl1_ui, Wl1_iu]))
    agg1, _ = _sc_agg(t1, srcs, dsts)
    zd = _epilogue(agg1, cnt, hd,
                   jnp.stack([Wr1_ui, Wr1_iu]),
                   jnp.stack([bl1_ui, bl1_iu]), relu=False)

    dec_idx = jnp.concatenate([
        edge_label_index[0] + N,
        edge_label_index[1],
        (jnp.arange(GPAD - 2 * EL, dtype=jnp.int32) % N),
    ])
    g = _sc_gather(zd, dec_idx)
    return _mlp(g, edge_attr, W1[0:D], W1[D:2 * D], W1[2 * D:], b1[None],
                W2, b2[None], W3, b3[None])


# speedup vs baseline: 6.5042x; 1.9391x over previous
import functools

import jax
import jax.numpy as jnp
from jax import lax
from jax.experimental import pallas as pl
from jax.experimental.pallas import tpu as pltpu
from jax.experimental.pallas import tpu_sc as plsc

N = 10000
ND = 2 * N
E = 320000
E_ALL = 2 * E
D = 128
EL = 100000
DE = 16

NC = 2
NS = 16
NW = NC * NS

CH = 128
EPT = 20480
NCHUNK = EPT // CH
EPC = NS * EPT
RPT = 640
NPAD = NS * RPT

GPAD = 229376
GPT = GPAD // NW
GCH = 128
GNCH = GPT // GCH

_f32 = jnp.float32


@functools.cache
def _build_sc_agg(with_cnt):
    mesh = plsc.VectorSubcoreMesh(core_axis_name="c", subcore_axis_name="s")
    outs = [jax.ShapeDtypeStruct((ND, D), _f32)]
    if with_cnt:
        outs.append(jax.ShapeDtypeStruct((ND,), _f32))

    @functools.partial(
        pl.kernel,
        mesh=mesh,
        out_type=outs,
        scratch_types=[
            pltpu.VMEM((NCHUNK // 4, CH), jnp.int32),
            pltpu.VMEM((NCHUNK // 4, CH), jnp.int32),
            pltpu.VMEM((CH, D), _f32),
            pltpu.VMEM((CH, D), _f32),
            pltpu.VMEM((CH,), _f32),
            pltpu.VMEM((RPT,), _f32),
            pltpu.VMEM_SHARED((NPAD, D), _f32),
            pltpu.VMEM_SHARED((NPAD,), _f32),
            pltpu.SemaphoreType.DMA,
            pltpu.SemaphoreType.DMA,
            pltpu.SemaphoreType.DMA,
            pltpu.SemaphoreType.DMA,
            pltpu.SemaphoreType.DMA,
            pltpu.SemaphoreType.DMA,
        ],
    )
    def sc_agg(table, srcs2d, dsts2d, *refs):
        if with_cnt:
            (agg_out, cnt_out, sidx2d, didx2d, rows0, rows1, onev, cvec,
             sagg, scnt, semi, sg0, sg1, ss0, ss1, smc) = refs
        else:
            (agg_out, sidx2d, didx2d, rows0, rows1, onev, cvec,
             sagg, scnt, semi, sg0, sg1, ss0, ss1, smc) = refs
            cnt_out = None
        c = lax.axis_index("c")
        s = lax.axis_index("s")
        NST = 4
        CPS = NCHUNK // NST

        tb = (c * NS + s) * NCHUNK
        pltpu.async_copy(srcs2d.at[pl.ds(tb, CPS)], sidx2d, semi)
        pltpu.async_copy(dsts2d.at[pl.ds(tb, CPS)], didx2d, semi)

        def zrow(j, carry):
            for g in range(D // 16):
                rows0[j, pl.ds(g * 16, 16)] = jnp.zeros((16,), _f32)
            return carry

        lax.fori_loop(0, CH, zrow, 0)
        for r in range(RPT // CH):
            pltpu.sync_copy(rows0, sagg.at[pl.ds(s * RPT + r * CH, CH)])
        if with_cnt:
            def zvec(j, carry):
                cvec[pl.ds(j * 16, 16)] = jnp.zeros((16,), _f32)
                return carry

            lax.fori_loop(0, RPT // 16, zvec, 0)

            def fill_ones(j, carry):
                onev[pl.ds(j * 16, 16)] = jnp.ones((16,), _f32)
                return carry

            lax.fori_loop(0, CH // 16, fill_ones, 0)
            pltpu.sync_copy(cvec, scnt.at[pl.ds(s * RPT, RPT)])

        pltpu.make_async_copy(srcs2d.at[pl.ds(tb, CPS)], sidx2d, semi).wait()
        pltpu.make_async_copy(dsts2d.at[pl.ds(tb, CPS)], didx2d, semi).wait()
        plsc.subcore_barrier()

        bufs = (rows0, rows1)
        gsems = (sg0, sg1)
        ssems = (ss0, ss1)

        def g_start(j, b):
            pltpu.async_copy(table.at[sidx2d.at[j]], bufs[b], gsems[b])

        def g_wait(j, b):
            pltpu.make_async_copy(table.at[sidx2d.at[j]], bufs[b],
                                  gsems[b]).wait()

        def s_start(j, b):
            pltpu.async_copy(bufs[b], sagg.at[didx2d.at[j]], ssems[b],
                             add=True)
            if with_cnt:
                pltpu.async_copy(onev, scnt.at[didx2d.at[j]], smc, add=True)

        def s_wait(j, b):
            pltpu.make_async_copy(bufs[b], sagg.at[didx2d.at[j]],
                                  ssems[b]).wait()

        def body(i, carry):
            j0 = 2 * i
            j1 = j0 + 1

            @pl.when(i > 0)
            def _():
                s_wait(j0 - 1, 1)

            g_start(j1, 1)
            g_wait(j0, 0)
            s_start(j0, 0)

            @pl.when(i < CPS // 2 - 1)
            def _():
                s_wait(j0, 0)
                g_start(j0 + 2, 0)

            g_wait(j1, 1)
            s_start(j1, 1)
            return carry

        for st in range(NST):
            if st > 0:
                pltpu.async_copy(srcs2d.at[pl.ds(tb + st * CPS, CPS)],
                                 sidx2d, semi)
                pltpu.async_copy(dsts2d.at[pl.ds(tb + st * CPS, CPS)],
                                 didx2d, semi)
                pltpu.make_async_copy(srcs2d.at[pl.ds(tb + st * CPS, CPS)],
                                      sidx2d, semi).wait()
                pltpu.make_async_copy(dsts2d.at[pl.ds(tb + st * CPS, CPS)],
                                      didx2d, semi).wait()
            g_start(0, 0)
            lax.fori_loop(0, CPS // 2, body, 0)
            s_wait(CPS - 2, 0)
            s_wait(CPS - 1, 1)
            if with_cnt:
                def drain(j, carry):
                    pltpu.make_async_copy(onev, scnt.at[didx2d.at[0]],
                                          smc).wait()
                    return carry

                lax.fori_loop(0, CPS, drain, 0)
        plsc.subcore_barrier()

        ob = c * N + s * RPT
        last = N - (NS - 1) * RPT

        if with_cnt:
            pltpu.sync_copy(scnt.at[pl.ds(s * RPT, RPT)], cvec)

        @pl.when(s < NS - 1)
        def _():
            pltpu.sync_copy(sagg.at[pl.ds(s * RPT, RPT)],
                            agg_out.at[pl.ds(ob, RPT)])
            if with_cnt:
                pltpu.sync_copy(cvec, cnt_out.at[pl.ds(ob, RPT)])

        @pl.when(s == NS - 1)
        def _():
            pltpu.sync_copy(sagg.at[pl.ds((NS - 1) * RPT, last)],
                            agg_out.at[pl.ds(c * N + (NS - 1) * RPT, last)])
            if with_cnt:
                pltpu.sync_copy(cvec.at[pl.ds(0, last)],
                                cnt_out.at[pl.ds(c * N + (NS - 1) * RPT,
                                                 last)])

    return sc_agg


def _sc_agg(table, srcs2d, dsts2d, with_cnt):
    out = _build_sc_agg(with_cnt)(table, srcs2d, dsts2d)
    if with_cnt:
        return out
    return out[0] if isinstance(out, (list, tuple)) else out


@functools.cache
def _build_sc_gather():
    mesh = plsc.VectorSubcoreMesh(core_axis_name="c", subcore_axis_name="s")

    @functools.partial(
        pl.kernel,
        mesh=mesh,
        out_type=jax.ShapeDtypeStruct((GPAD, D), _f32),
        scratch_types=[
            pltpu.VMEM((GNCH, GCH), jnp.int32),
            pltpu.VMEM((GCH, D), _f32),
            pltpu.VMEM((GCH, D), _f32),
            pltpu.SemaphoreType.DMA,
            pltpu.SemaphoreType.DMA,
            pltpu.SemaphoreType.DMA,
            pltpu.SemaphoreType.DMA,
            pltpu.SemaphoreType.DMA,
        ],
    )
    def sc_gather(z, idx2d, out, vidx, rows0, rows1, semi, sg0, sg1, so0, so1):
        c = lax.axis_index("c")
        s = lax.axis_index("s")
        w = c * NS + s
        tb = w * GNCH
        base = w * GPT
        pltpu.sync_copy(idx2d.at[pl.ds(tb, GNCH)], vidx)

        bufs = (rows0, rows1)
        gsems = (sg0, sg1)
        osems = (so0, so1)

        def g_start(j, b):
            pltpu.async_copy(z.at[vidx.at[j]], bufs[b], gsems[b])

        def g_wait(j, b):
            pltpu.make_async_copy(z.at[vidx.at[j]], bufs[b], gsems[b]).wait()

        def o_start(j, b):
            pltpu.async_copy(bufs[b], out.at[pl.ds(base + j * GCH, GCH)],
                             osems[b])

        def o_wait(j, b):
            pltpu.make_async_copy(bufs[b], out.at[pl.ds(base + j * GCH, GCH)],
                                  osems[b]).wait()

        g_start(0, 0)

        def body(i, carry):
            j0 = 2 * i
            j1 = j0 + 1

            @pl.when(i > 0)
            def _():
                o_wait(j0 - 1, 1)

            g_start(j1, 1)
            g_wait(j0, 0)
            o_start(j0, 0)

            @pl.when(i < GNCH // 2 - 1)
            def _():
                o_wait(j0, 0)
                g_start(j0 + 2, 0)

            g_wait(j1, 1)
            o_start(j1, 1)
            return carry

        lax.fori_loop(0, GNCH // 2, body, 0)
        o_wait(GNCH - 2, 0)
        o_wait(GNCH - 1, 1)

    return sc_gather


def _sc_gather(*args):
    return _build_sc_gather()(*args)


BR = 400
NB = ND // BR
HB = NB // 2

def _transform_body(x_ref, w_ref, o_ref):
    o_ref[...] = jnp.dot(x_ref[...], w_ref[0], preferred_element_type=_f32)


def _transform(xd, wstack):
    return pl.pallas_call(
        _transform_body,
        grid=(NB,),
        in_specs=[
            pl.BlockSpec((BR, D), lambda i: ((i + HB) % NB, 0)),
            pl.BlockSpec((1, D, D), lambda i: (i // HB, 0, 0)),
        ],
        out_specs=pl.BlockSpec((BR, D), lambda i: (i, 0)),
        out_shape=jax.ShapeDtypeStruct((ND, D), _f32),
    )(xd, wstack)


def _epilogue_body(relu, agg_ref, cnt_ref, x_ref, w_ref, b_ref, o_ref):
    cnt = jnp.maximum(cnt_ref[...], 1.0)
    h = agg_ref[...] / cnt + jnp.dot(
        x_ref[...], w_ref[0], preferred_element_type=_f32) + b_ref[0, 0]
    if relu:
        h = jnp.maximum(h, 0.0)
    o_ref[...] = h


def _epilogue(agg, cnt, xd, wstack, bstack, relu):
    return pl.pallas_call(
        functools.partial(_epilogue_body, relu),
        grid=(NB,),
        in_specs=[
            pl.BlockSpec((BR, D), lambda i: (i, 0)),
            pl.BlockSpec((BR, 1), lambda i: (i, 0)),
            pl.BlockSpec((BR, D), lambda i: (i, 0)),
            pl.BlockSpec((1, D, D), lambda i: (i // HB, 0, 0)),
            pl.BlockSpec((1, 1, D), lambda i: (i // HB, 0, 0)),
        ],
        out_specs=pl.BlockSpec((BR, D), lambda i: (i, 0)),
        out_shape=jax.ShapeDtypeStruct((ND, D), _f32),
    )(agg, cnt[:, None], xd, wstack, bstack[:, None])


MR = 400
MB = EL // MR


def _mlp_body(gu_ref, gi_ref, ea_ref, w1u_ref, w1i_ref, w1e_ref, b1_ref,
              w2_ref, b2_ref, w3_ref, b3_ref, o_ref):
    z = (jnp.dot(gu_ref[...], w1u_ref[...], preferred_element_type=_f32)
         + jnp.dot(gi_ref[...], w1i_ref[...], preferred_element_type=_f32)
         + jnp.dot(ea_ref[...], w1e_ref[...], preferred_element_type=_f32)
         + b1_ref[...])
    z = jnp.maximum(z, 0.0)
    z = jnp.maximum(jnp.dot(z, w2_ref[...], preferred_element_type=_f32)
                    + b2_ref[...], 0.0)
    o_ref[...] = jnp.dot(z, w3_ref[...], preferred_element_type=_f32) + b3_ref[...]


def _mlp(g, ea, w1u, w1i, w1e, b1, w2, b2, w3, b3):
    full = lambda i: (0, 0)
    return pl.pallas_call(
        _mlp_body,
        grid=(MB,),
        in_specs=[
            pl.BlockSpec((MR, D), lambda i: (i, 0)),
            pl.BlockSpec((MR, D), lambda i: (i + MB, 0)),
            pl.BlockSpec((MR, DE), lambda i: (i, 0)),
            pl.BlockSpec((D, D), full),
            pl.BlockSpec((D, D), full),
            pl.BlockSpec((DE, D), full),
            pl.BlockSpec((1, D), full),
            pl.BlockSpec((D, D), full),
            pl.BlockSpec((1, D), full),
            pl.BlockSpec((D, 2), full),
            pl.BlockSpec((1, 2), full),
        ],
        out_specs=pl.BlockSpec((MR, 2), lambda i: (i, 0)),
        out_shape=jax.ShapeDtypeStruct((EL, 2), _f32),
    )(g, g, ea, w1u, w1i, w1e, b1, w2, b2, w3, b3)


def kernel(x_user, x_item, edge_attr, Wl0_ui, bl0_ui, Wr0_ui, Wl0_iu, bl0_iu,
           Wr0_iu, Wl1_ui, bl1_ui, Wr1_ui, Wl1_iu, bl1_iu, Wr1_iu, W1, b1,
           W2, b2, W3, b3, ei_ui, ei_iu, edge_label_index):
    xd = jnp.concatenate([x_item, x_user], axis=0)
    npad_e = EPC - E
    pad_src = jnp.arange(npad_e, dtype=jnp.int32) % N
    pad_dst = N + (jnp.arange(npad_e, dtype=jnp.int32) % (NPAD - N))
    srcs2d = jnp.concatenate([
        ei_ui[0], pad_src, ei_iu[0] + N, pad_src,
    ]).reshape(2 * EPC // CH, CH)
    dsts2d = jnp.concatenate([
        ei_ui[1], pad_dst, ei_iu[1], pad_dst,
    ]).reshape(2 * EPC // CH, CH)

    t0 = _transform(xd, jnp.stack([Wl0_ui, Wl0_iu]))
    agg0, cnt = _sc_agg(t0, srcs2d, dsts2d, True)
    hd = _epilogue(agg0, cnt, xd,
                   jnp.stack([Wr0_ui, Wr0_iu]),
                   jnp.stack([bl0_ui, bl0_iu]), relu=True)
    t1 = _transform(hd, jnp.stack([Wl1_ui, Wl1_iu]))
    agg1 = _sc_agg(t1, srcs2d, dsts2d, False)
    zd = _epilogue(agg1, cnt, hd,
                   jnp.stack([Wr1_ui, Wr1_iu]),
                   jnp.stack([bl1_ui, bl1_iu]), relu=False)

    dec_idx = jnp.concatenate([
        edge_label_index[0] + N,
        edge_label_index[1],
        (jnp.arange(GPAD - 2 * EL, dtype=jnp.int32) % N),
    ]).reshape(GPAD // GCH, GCH)
    g = _sc_gather(zd, dec_idx)
    return _mlp(g, edge_attr, W1[0:D], W1[D:2 * D], W1[2 * D:], b1[None],
                W2, b2[None], W3, b3[None])


# speedup vs baseline: 6.6848x; 1.0278x over previous
import functools

import jax
import jax.numpy as jnp
from jax import lax
from jax.experimental import pallas as pl
from jax.experimental.pallas import tpu as pltpu
from jax.experimental.pallas import tpu_sc as plsc

N = 10000
ND = 2 * N
E = 320000
E_ALL = 2 * E
D = 128
EL = 100000
DE = 16

NC = 2
NS = 16
NW = NC * NS

CH = 128
EPT = 20480
NCHUNK = EPT // CH
EPC = NS * EPT
RPT = 640
NPAD = NS * RPT

GPAD = 229376
GPT = GPAD // NW
GCH = 128
GNCH = GPT // GCH

_f32 = jnp.float32


@functools.cache
def _build_sc_agg(with_cnt):
    mesh = plsc.VectorSubcoreMesh(core_axis_name="c", subcore_axis_name="s")
    outs = [jax.ShapeDtypeStruct((ND, D), _f32)]
    if with_cnt:
        outs.append(jax.ShapeDtypeStruct((ND,), _f32))

    @functools.partial(
        pl.kernel,
        mesh=mesh,
        out_type=outs,
        scratch_types=[
            pltpu.VMEM((NCHUNK // 4, CH), jnp.int32),
            pltpu.VMEM((NCHUNK // 4, CH), jnp.int32),
            pltpu.VMEM((CH, D), _f32),
            pltpu.VMEM((CH, D), _f32),
            pltpu.VMEM((CH,), _f32),
            pltpu.VMEM((RPT,), _f32),
            pltpu.VMEM_SHARED((NPAD, D), _f32),
            pltpu.VMEM_SHARED((NPAD,), _f32),
            pltpu.SemaphoreType.DMA,
            pltpu.SemaphoreType.DMA,
            pltpu.SemaphoreType.DMA,
            pltpu.SemaphoreType.DMA,
            pltpu.SemaphoreType.DMA,
            pltpu.SemaphoreType.DMA,
        ],
    )
    def sc_agg(table, srcs2d, dsts2d, *refs):
        if with_cnt:
            (agg_out, cnt_out, sidx2d, didx2d, rows0, rows1, onev, cvec,
             sagg, scnt, semi, sg0, sg1, ss0, ss1, smc) = refs
        else:
            (agg_out, sidx2d, didx2d, rows0, rows1, onev, cvec,
             sagg, scnt, semi, sg0, sg1, ss0, ss1, smc) = refs
            cnt_out = None
        c = lax.axis_index("c")
        s = lax.axis_index("s")
        NST = 4
        CPS = NCHUNK // NST

        tb = (c * NS + s) * NCHUNK
        pltpu.async_copy(srcs2d.at[pl.ds(tb, CPS)], sidx2d, semi)
        pltpu.async_copy(dsts2d.at[pl.ds(tb, CPS)], didx2d, semi)

        def zrow(j, carry):
            for g in range(D // 16):
                rows0[j, pl.ds(g * 16, 16)] = jnp.zeros((16,), _f32)
            return carry

        lax.fori_loop(0, CH, zrow, 0)
        for r in range(RPT // CH):
            pltpu.sync_copy(rows0, sagg.at[pl.ds(s * RPT + r * CH, CH)])
        if with_cnt:
            def zvec(j, carry):
                cvec[pl.ds(j * 16, 16)] = jnp.zeros((16,), _f32)
                return carry

            lax.fori_loop(0, RPT // 16, zvec, 0)

            def fill_ones(j, carry):
                onev[pl.ds(j * 16, 16)] = jnp.ones((16,), _f32)
                return carry

            lax.fori_loop(0, CH // 16, fill_ones, 0)
            pltpu.sync_copy(cvec, scnt.at[pl.ds(s * RPT, RPT)])

        pltpu.make_async_copy(srcs2d.at[pl.ds(tb, CPS)], sidx2d, semi).wait()
        pltpu.make_async_copy(dsts2d.at[pl.ds(tb, CPS)], didx2d, semi).wait()
        plsc.subcore_barrier()

        bufs = (rows0, rows1)
        gsems = (sg0, sg1)
        ssems = (ss0, ss1)

        def g_start(j, b):
            pltpu.async_copy(table.at[sidx2d.at[j]], bufs[b], gsems[b])

        def g_wait(j, b):
            pltpu.make_async_copy(table.at[sidx2d.at[j]], bufs[b],
                                  gsems[b]).wait()

        def s_start(j, b):
            pltpu.async_copy(bufs[b], sagg.at[didx2d.at[j]], ssems[b],
                             add=True)
            if with_cnt:
                pltpu.async_copy(onev, scnt.at[didx2d.at[j]], smc, add=True)

        def c_wait():
            if with_cnt:
                pltpu.make_async_copy(onev, scnt.at[didx2d.at[0]], smc).wait()

        def s_wait(j, b):
            pltpu.make_async_copy(bufs[b], sagg.at[didx2d.at[j]],
                                  ssems[b]).wait()

        def body(i, carry):
            j0 = 2 * i
            j1 = j0 + 1

            @pl.when(i > 0)
            def _():
                s_wait(j0 - 1, 1)
                c_wait()
                c_wait()

            g_start(j1, 1)
            g_wait(j0, 0)
            s_start(j0, 0)

            @pl.when(i < CPS // 2 - 1)
            def _():
                s_wait(j0, 0)
                g_start(j0 + 2, 0)

            g_wait(j1, 1)
            s_start(j1, 1)
            return carry

        for st in range(NST):
            if st > 0:
                pltpu.async_copy(srcs2d.at[pl.ds(tb + st * CPS, CPS)],
                                 sidx2d, semi)
                pltpu.async_copy(dsts2d.at[pl.ds(tb + st * CPS, CPS)],
                                 didx2d, semi)
                pltpu.make_async_copy(srcs2d.at[pl.ds(tb + st * CPS, CPS)],
                                      sidx2d, semi).wait()
                pltpu.make_async_copy(dsts2d.at[pl.ds(tb + st * CPS, CPS)],
                                      didx2d, semi).wait()
            g_start(0, 0)
            lax.fori_loop(0, CPS // 2, body, 0)
            s_wait(CPS - 2, 0)
            s_wait(CPS - 1, 1)
            c_wait()
            c_wait()
        plsc.subcore_barrier()

        ob = c * N + s * RPT
        last = N - (NS - 1) * RPT

        if with_cnt:
            pltpu.sync_copy(scnt.at[pl.ds(s * RPT, RPT)], cvec)

        @pl.when(s < NS - 1)
        def _():
            pltpu.sync_copy(sagg.at[pl.ds(s * RPT, RPT)],
                            agg_out.at[pl.ds(ob, RPT)])
            if with_cnt:
                pltpu.sync_copy(cvec, cnt_out.at[pl.ds(ob, RPT)])

        @pl.when(s == NS - 1)
        def _():
            pltpu.sync_copy(sagg.at[pl.ds((NS - 1) * RPT, last)],
                            agg_out.at[pl.ds(c * N + (NS - 1) * RPT, last)])
            if with_cnt:
                pltpu.sync_copy(cvec.at[pl.ds(0, last)],
                                cnt_out.at[pl.ds(c * N + (NS - 1) * RPT,
                                                 last)])

    return sc_agg


def _sc_agg(table, srcs2d, dsts2d, with_cnt):
    out = _build_sc_agg(with_cnt)(table, srcs2d, dsts2d)
    if with_cnt:
        return out
    return out[0] if isinstance(out, (list, tuple)) else out


@functools.cache
def _build_sc_gather():
    mesh = plsc.VectorSubcoreMesh(core_axis_name="c", subcore_axis_name="s")

    @functools.partial(
        pl.kernel,
        mesh=mesh,
        out_type=jax.ShapeDtypeStruct((GPAD, D), _f32),
        scratch_types=[
            pltpu.VMEM((GNCH, GCH), jnp.int32),
            pltpu.VMEM((GCH, D), _f32),
            pltpu.VMEM((GCH, D), _f32),
            pltpu.SemaphoreType.DMA,
            pltpu.SemaphoreType.DMA,
            pltpu.SemaphoreType.DMA,
            pltpu.SemaphoreType.DMA,
            pltpu.SemaphoreType.DMA,
        ],
    )
    def sc_gather(z, idx2d, out, vidx, rows0, rows1, semi, sg0, sg1, so0, so1):
        c = lax.axis_index("c")
        s = lax.axis_index("s")
        w = c * NS + s
        tb = w * GNCH
        base = w * GPT
        pltpu.sync_copy(idx2d.at[pl.ds(tb, GNCH)], vidx)

        bufs = (rows0, rows1)
        gsems = (sg0, sg1)
        osems = (so0, so1)

        def g_start(j, b):
            pltpu.async_copy(z.at[vidx.at[j]], bufs[b], gsems[b])

        def g_wait(j, b):
            pltpu.make_async_copy(z.at[vidx.at[j]], bufs[b], gsems[b]).wait()

        def o_start(j, b):
            pltpu.async_copy(bufs[b], out.at[pl.ds(base + j * GCH, GCH)],
                             osems[b])

        def o_wait(j, b):
            pltpu.make_async_copy(bufs[b], out.at[pl.ds(base + j * GCH, GCH)],
                                  osems[b]).wait()

        g_start(0, 0)

        def body(i, carry):
            j0 = 2 * i
            j1 = j0 + 1

            @pl.when(i > 0)
            def _():
                o_wait(j0 - 1, 1)

            g_start(j1, 1)
            g_wait(j0, 0)
            o_start(j0, 0)

            @pl.when(i < GNCH // 2 - 1)
            def _():
                o_wait(j0, 0)
                g_start(j0 + 2, 0)

            g_wait(j1, 1)
            o_start(j1, 1)
            return carry

        lax.fori_loop(0, GNCH // 2, body, 0)
        o_wait(GNCH - 2, 0)
        o_wait(GNCH - 1, 1)

    return sc_gather


def _sc_gather(*args):
    return _build_sc_gather()(*args)


BR = 400
NB = ND // BR
HB = NB // 2

def _transform_body(x_ref, w_ref, o_ref):
    o_ref[...] = jnp.dot(x_ref[...], w_ref[0], preferred_element_type=_f32)


def _transform(xd, wstack):
    return pl.pallas_call(
        _transform_body,
        grid=(NB,),
        in_specs=[
            pl.BlockSpec((BR, D), lambda i: ((i + HB) % NB, 0)),
            pl.BlockSpec((1, D, D), lambda i: (i // HB, 0, 0)),
        ],
        out_specs=pl.BlockSpec((BR, D), lambda i: (i, 0)),
        out_shape=jax.ShapeDtypeStruct((ND, D), _f32),
    )(xd, wstack)


def _epilogue_body(relu, agg_ref, cnt_ref, x_ref, w_ref, b_ref, o_ref):
    cnt = jnp.maximum(cnt_ref[...], 1.0)
    h = agg_ref[...] / cnt + jnp.dot(
        x_ref[...], w_ref[0], preferred_element_type=_f32) + b_ref[0, 0]
    if relu:
        h = jnp.maximum(h, 0.0)
    o_ref[...] = h


def _epilogue(agg, cnt, xd, wstack, bstack, relu):
    return pl.pallas_call(
        functools.partial(_epilogue_body, relu),
        grid=(NB,),
        in_specs=[
            pl.BlockSpec((BR, D), lambda i: (i, 0)),
            pl.BlockSpec((BR, 1), lambda i: (i, 0)),
            pl.BlockSpec((BR, D), lambda i: (i, 0)),
            pl.BlockSpec((1, D, D), lambda i: (i // HB, 0, 0)),
            pl.BlockSpec((1, 1, D), lambda i: (i // HB, 0, 0)),
        ],
        out_specs=pl.BlockSpec((BR, D), lambda i: (i, 0)),
        out_shape=jax.ShapeDtypeStruct((ND, D), _f32),
    )(agg, cnt[:, None], xd, wstack, bstack[:, None])


def _epi_trans_body(agg_ref, cnt_ref, x_ref, wr_ref, b_ref, wl_ref,
                    h_ref, t_ref):
    cnt = jnp.maximum(cnt_ref[...], 1.0)
    h = agg_ref[...] / cnt + jnp.dot(
        x_ref[...], wr_ref[0], preferred_element_type=_f32) + b_ref[0, 0]
    h = jnp.maximum(h, 0.0)
    h_ref[...] = h
    t_ref[...] = jnp.dot(h, wl_ref[0], preferred_element_type=_f32)


def _epi_trans(agg, cnt, xd, wrstack, bstack, wlstack):
    return pl.pallas_call(
        _epi_trans_body,
        grid=(NB,),
        in_specs=[
            pl.BlockSpec((BR, D), lambda i: (i, 0)),
            pl.BlockSpec((BR, 1), lambda i: (i, 0)),
            pl.BlockSpec((BR, D), lambda i: (i, 0)),
            pl.BlockSpec((1, D, D), lambda i: (i // HB, 0, 0)),
            pl.BlockSpec((1, 1, D), lambda i: (i // HB, 0, 0)),
            pl.BlockSpec((1, D, D), lambda i: (1 - i // HB, 0, 0)),
        ],
        out_specs=[
            pl.BlockSpec((BR, D), lambda i: (i, 0)),
            pl.BlockSpec((BR, D), lambda i: ((i + HB) % NB, 0)),
        ],
        out_shape=[jax.ShapeDtypeStruct((ND, D), _f32),
                   jax.ShapeDtypeStruct((ND, D), _f32)],
    )(agg, cnt[:, None], xd, wrstack, bstack[:, None], wlstack)


MR = 400
MB = EL // MR


def _mlp_body(gu_ref, gi_ref, ea_ref, w1u_ref, w1i_ref, w1e_ref, b1_ref,
              w2_ref, b2_ref, w3_ref, b3_ref, o_ref):
    z = (jnp.dot(gu_ref[...], w1u_ref[...], preferred_element_type=_f32)
         + jnp.dot(gi_ref[...], w1i_ref[...], preferred_element_type=_f32)
         + jnp.dot(ea_ref[...], w1e_ref[...], preferred_element_type=_f32)
         + b1_ref[...])
    z = jnp.maximum(z, 0.0)
    z = jnp.maximum(jnp.dot(z, w2_ref[...], preferred_element_type=_f32)
                    + b2_ref[...], 0.0)
    o_ref[...] = jnp.dot(z, w3_ref[...], preferred_element_type=_f32) + b3_ref[...]


def _mlp(g, ea, w1u, w1i, w1e, b1, w2, b2, w3, b3):
    full = lambda i: (0, 0)
    return pl.pallas_call(
        _mlp_body,
        grid=(MB,),
        in_specs=[
            pl.BlockSpec((MR, D), lambda i: (i, 0)),
            pl.BlockSpec((MR, D), lambda i: (i + MB, 0)),
            pl.BlockSpec((MR, DE), lambda i: (i, 0)),
            pl.BlockSpec((D, D), full),
            pl.BlockSpec((D, D), full),
            pl.BlockSpec((DE, D), full),
            pl.BlockSpec((1, D), full),
            pl.BlockSpec((D, D), full),
            pl.BlockSpec((1, D), full),
            pl.BlockSpec((D, 2), full),
            pl.BlockSpec((1, 2), full),
        ],
        out_specs=pl.BlockSpec((MR, 2), lambda i: (i, 0)),
        out_shape=jax.ShapeDtypeStruct((EL, 2), _f32),
    )(g, g, ea, w1u, w1i, w1e, b1, w2, b2, w3, b3)


def kernel(x_user, x_item, edge_attr, Wl0_ui, bl0_ui, Wr0_ui, Wl0_iu, bl0_iu,
           Wr0_iu, Wl1_ui, bl1_ui, Wr1_ui, Wl1_iu, bl1_iu, Wr1_iu, W1, b1,
           W2, b2, W3, b3, ei_ui, ei_iu, edge_label_index):
    xd = jnp.concatenate([x_item, x_user], axis=0)
    npad_e = EPC - E
    pad_src = jnp.arange(npad_e, dtype=jnp.int32) % N
    pad_dst = N + (jnp.arange(npad_e, dtype=jnp.int32) % (NPAD - N))
    srcs2d = jnp.concatenate([
        ei_ui[0], pad_src, ei_iu[0] + N, pad_src,
    ]).reshape(2 * EPC // CH, CH)
    dsts2d = jnp.concatenate([
        ei_ui[1], pad_dst, ei_iu[1], pad_dst,
    ]).reshape(2 * EPC // CH, CH)

    t0 = _transform(xd, jnp.stack([Wl0_ui, Wl0_iu]))
    agg0, cnt = _sc_agg(t0, srcs2d, dsts2d, True)
    hd, t1 = _epi_trans(agg0, cnt, xd,
                        jnp.stack([Wr0_ui, Wr0_iu]),
                        jnp.stack([bl0_ui, bl0_iu]),
                        jnp.stack([Wl1_ui, Wl1_iu]))
    agg1 = _sc_agg(t1, srcs2d, dsts2d, False)
    zd = _epilogue(agg1, cnt, hd,
                   jnp.stack([Wr1_ui, Wr1_iu]),
                   jnp.stack([bl1_ui, bl1_iu]), relu=False)

    dec_idx = jnp.concatenate([
        edge_label_index[0] + N,
        edge_label_index[1],
        (jnp.arange(GPAD - 2 * EL, dtype=jnp.int32) % N),
    ]).reshape(GPAD // GCH, GCH)
    g = _sc_gather(zd, dec_idx)
    return _mlp(g, edge_attr, W1[0:D], W1[D:2 * D], W1[2 * D:], b1[None],
                W2, b2[None], W3, b3[None])


# speedup vs baseline: 8.3847x; 1.2543x over previous
import functools

import jax
import jax.numpy as jnp
from jax import lax
from jax.experimental import pallas as pl
from jax.experimental.pallas import tpu as pltpu
from jax.experimental.pallas import tpu_sc as plsc

N = 10000
ND = 2 * N
E = 320000
E_ALL = 2 * E
D = 128
EL = 100000
DE = 16

NC = 2
NS = 16
NW = NC * NS

CH = 128
EPT = 20480
NCHUNK = EPT // CH
EPC = NS * EPT
RPT = 640
NPAD = NS * RPT

GPAD = 229376
GPT = GPAD // NW
GCH = 128
GNCH = GPT // GCH

_f32 = jnp.float32


@functools.cache
def _build_sc_agg(with_cnt):
    mesh = plsc.VectorSubcoreMesh(core_axis_name="c", subcore_axis_name="s")
    outs = [jax.ShapeDtypeStruct((ND, D), _f32)]
    if with_cnt:
        outs.append(jax.ShapeDtypeStruct((ND,), _f32))

    @functools.partial(
        pl.kernel,
        mesh=mesh,
        out_type=outs,
        scratch_types=[
            pltpu.VMEM((NCHUNK // 4, CH), jnp.int32),
            pltpu.VMEM((NCHUNK // 4, CH), jnp.int32),
            pltpu.VMEM((CH, D), _f32),
            pltpu.VMEM((CH, D), _f32),
            pltpu.VMEM((CH,), _f32),
            pltpu.VMEM((RPT,), _f32),
            pltpu.VMEM_SHARED((NPAD, D), _f32),
            pltpu.VMEM_SHARED((NPAD,), _f32),
            pltpu.SemaphoreType.DMA,
            pltpu.SemaphoreType.DMA,
            pltpu.SemaphoreType.DMA,
            pltpu.SemaphoreType.DMA,
            pltpu.SemaphoreType.DMA,
            pltpu.SemaphoreType.DMA,
        ],
    )
    def sc_agg(table, srcs2d, dsts2d, *refs):
        if with_cnt:
            (agg_out, cnt_out, sidx2d, didx2d, rows0, rows1, onev, cvec,
             sagg, scnt, semi, sg0, sg1, ss0, ss1, smc) = refs
        else:
            (agg_out, sidx2d, didx2d, rows0, rows1, onev, cvec,
             sagg, scnt, semi, sg0, sg1, ss0, ss1, smc) = refs
            cnt_out = None
        c = lax.axis_index("c")
        s = lax.axis_index("s")
        NST = 4
        CPS = NCHUNK // NST

        tb = (c * NS + s) * NCHUNK
        pltpu.async_copy(srcs2d.at[pl.ds(tb, CPS)], sidx2d, semi)
        pltpu.async_copy(dsts2d.at[pl.ds(tb, CPS)], didx2d, semi)

        def zrow(j, carry):
            for g in range(D // 16):
                rows0[j, pl.ds(g * 16, 16)] = jnp.zeros((16,), _f32)
            return carry

        lax.fori_loop(0, CH, zrow, 0)
        for r in range(RPT // CH):
            pltpu.sync_copy(rows0, sagg.at[pl.ds(s * RPT + r * CH, CH)])
        if with_cnt:
            def zvec(j, carry):
                cvec[pl.ds(j * 16, 16)] = jnp.zeros((16,), _f32)
                return carry

            lax.fori_loop(0, RPT // 16, zvec, 0)

            def fill_ones(j, carry):
                onev[pl.ds(j * 16, 16)] = jnp.ones((16,), _f32)
                return carry

            lax.fori_loop(0, CH // 16, fill_ones, 0)
            pltpu.sync_copy(cvec, scnt.at[pl.ds(s * RPT, RPT)])

        pltpu.make_async_copy(srcs2d.at[pl.ds(tb, CPS)], sidx2d, semi).wait()
        pltpu.make_async_copy(dsts2d.at[pl.ds(tb, CPS)], didx2d, semi).wait()
        plsc.subcore_barrier()

        bufs = (rows0, rows1)
        gsems = (sg0, sg1)
        ssems = (ss0, ss1)

        def g_start(j, b):
            pltpu.async_copy(table.at[sidx2d.at[j]], bufs[b], gsems[b])

        def g_wait(j, b):
            pltpu.make_async_copy(table.at[sidx2d.at[j]], bufs[b],
                                  gsems[b]).wait()

        def s_start(j, b):
            pltpu.async_copy(bufs[b], sagg.at[didx2d.at[j]], ssems[b],
                             add=True)
            if with_cnt:
                pltpu.async_copy(onev, scnt.at[didx2d.at[j]], smc, add=True)

        def c_wait():
            if with_cnt:
                pltpu.make_async_copy(onev, scnt.at[didx2d.at[0]], smc).wait()

        def s_wait(j, b):
            pltpu.make_async_copy(bufs[b], sagg.at[didx2d.at[j]],
                                  ssems[b]).wait()

        def body(i, carry):
            j0 = 2 * i
            j1 = j0 + 1

            @pl.when(i > 0)
            def _():
                s_wait(j0 - 1, 1)
                c_wait()
                c_wait()

            g_start(j1, 1)
            g_wait(j0, 0)
            s_start(j0, 0)

            @pl.when(i < CPS // 2 - 1)
            def _():
                s_wait(j0, 0)
                g_start(j0 + 2, 0)

            g_wait(j1, 1)
            s_start(j1, 1)
            return carry

        for st in range(NST):
            if st > 0:
                pltpu.async_copy(srcs2d.at[pl.ds(tb + st * CPS, CPS)],
                                 sidx2d, semi)
                pltpu.async_copy(dsts2d.at[pl.ds(tb + st * CPS, CPS)],
                                 didx2d, semi)
                pltpu.make_async_copy(srcs2d.at[pl.ds(tb + st * CPS, CPS)],
                                      sidx2d, semi).wait()
                pltpu.make_async_copy(dsts2d.at[pl.ds(tb + st * CPS, CPS)],
                                      didx2d, semi).wait()
            g_start(0, 0)
            lax.fori_loop(0, CPS // 2, body, 0)
            s_wait(CPS - 2, 0)
            s_wait(CPS - 1, 1)
            c_wait()
            c_wait()
        plsc.subcore_barrier()

        ob = c * N + s * RPT
        last = N - (NS - 1) * RPT

        if with_cnt:
            pltpu.sync_copy(scnt.at[pl.ds(s * RPT, RPT)], cvec)

        @pl.when(s < NS - 1)
        def _():
            pltpu.sync_copy(sagg.at[pl.ds(s * RPT, RPT)],
                            agg_out.at[pl.ds(ob, RPT)])
            if with_cnt:
                pltpu.sync_copy(cvec, cnt_out.at[pl.ds(ob, RPT)])

        @pl.when(s == NS - 1)
        def _():
            pltpu.sync_copy(sagg.at[pl.ds((NS - 1) * RPT, last)],
                            agg_out.at[pl.ds(c * N + (NS - 1) * RPT, last)])
            if with_cnt:
                pltpu.sync_copy(cvec.at[pl.ds(0, last)],
                                cnt_out.at[pl.ds(c * N + (NS - 1) * RPT,
                                                 last)])

    return sc_agg


def _sc_agg(table, srcs2d, dsts2d, with_cnt):
    out = _build_sc_agg(with_cnt)(table, srcs2d, dsts2d)
    if with_cnt:
        return out
    return out[0] if isinstance(out, (list, tuple)) else out


@functools.cache
def _build_sc_gather():
    mesh = plsc.VectorSubcoreMesh(core_axis_name="c", subcore_axis_name="s")

    @functools.partial(
        pl.kernel,
        mesh=mesh,
        out_type=jax.ShapeDtypeStruct((GPAD, D), _f32),
        scratch_types=[
            pltpu.VMEM((GNCH, GCH), jnp.int32),
            pltpu.VMEM((GCH, D), _f32),
            pltpu.VMEM((GCH, D), _f32),
            pltpu.SemaphoreType.DMA,
            pltpu.SemaphoreType.DMA,
            pltpu.SemaphoreType.DMA,
            pltpu.SemaphoreType.DMA,
            pltpu.SemaphoreType.DMA,
        ],
    )
    def sc_gather(z, idx2d, out, vidx, rows0, rows1, semi, sg0, sg1, so0, so1):
        c = lax.axis_index("c")
        s = lax.axis_index("s")
        w = c * NS + s
        tb = w * GNCH
        base = w * GPT
        pltpu.sync_copy(idx2d.at[pl.ds(tb, GNCH)], vidx)

        bufs = (rows0, rows1)
        gsems = (sg0, sg1)
        osems = (so0, so1)

        def g_start(j, b):
            pltpu.async_copy(z.at[vidx.at[j]], bufs[b], gsems[b])

        def g_wait(j, b):
            pltpu.make_async_copy(z.at[vidx.at[j]], bufs[b], gsems[b]).wait()

        def o_start(j, b):
            pltpu.async_copy(bufs[b], out.at[pl.ds(base + j * GCH, GCH)],
                             osems[b])

        def o_wait(j, b):
            pltpu.make_async_copy(bufs[b], out.at[pl.ds(base + j * GCH, GCH)],
                                  osems[b]).wait()

        g_start(0, 0)

        def body(i, carry):
            j0 = 2 * i
            j1 = j0 + 1

            @pl.when(i > 0)
            def _():
                o_wait(j0 - 1, 1)

            g_start(j1, 1)
            g_wait(j0, 0)
            o_start(j0, 0)

            @pl.when(i < GNCH // 2 - 1)
            def _():
                o_wait(j0, 0)
                g_start(j0 + 2, 0)

            g_wait(j1, 1)
            o_start(j1, 1)
            return carry

        lax.fori_loop(0, GNCH // 2, body, 0)
        o_wait(GNCH - 2, 0)
        o_wait(GNCH - 1, 1)

    return sc_gather


def _sc_gather(*args):
    return _build_sc_gather()(*args)


BR = 2000
NB = ND // BR
HB = NB // 2

def _transform_body(x_ref, w_ref, o_ref):
    o_ref[...] = jnp.dot(x_ref[...], w_ref[0], preferred_element_type=_f32)


def _transform(xd, wstack):
    return pl.pallas_call(
        _transform_body,
        grid=(NB,),
        in_specs=[
            pl.BlockSpec((BR, D), lambda i: ((i + HB) % NB, 0)),
            pl.BlockSpec((1, D, D), lambda i: (i // HB, 0, 0)),
        ],
        out_specs=pl.BlockSpec((BR, D), lambda i: (i, 0)),
        out_shape=jax.ShapeDtypeStruct((ND, D), _f32),
    )(xd, wstack)


def _epilogue_body(relu, agg_ref, cnt_ref, x_ref, w_ref, b_ref, o_ref):
    cnt = jnp.maximum(cnt_ref[...], 1.0)
    h = agg_ref[...] / cnt + jnp.dot(
        x_ref[...], w_ref[0], preferred_element_type=_f32) + b_ref[0, 0]
    if relu:
        h = jnp.maximum(h, 0.0)
    o_ref[...] = h


def _epilogue(agg, cnt, xd, wstack, bstack, relu):
    return pl.pallas_call(
        functools.partial(_epilogue_body, relu),
        grid=(NB,),
        in_specs=[
            pl.BlockSpec((BR, D), lambda i: (i, 0)),
            pl.BlockSpec((BR, 1), lambda i: (i, 0)),
            pl.BlockSpec((BR, D), lambda i: (i, 0)),
            pl.BlockSpec((1, D, D), lambda i: (i // HB, 0, 0)),
            pl.BlockSpec((1, 1, D), lambda i: (i // HB, 0, 0)),
        ],
        out_specs=pl.BlockSpec((BR, D), lambda i: (i, 0)),
        out_shape=jax.ShapeDtypeStruct((ND, D), _f32),
    )(agg, cnt[:, None], xd, wstack, bstack[:, None])


def _epi_trans_body(agg_ref, cnt_ref, x_ref, wr_ref, b_ref, wl_ref,
                    h_ref, t_ref):
    cnt = jnp.maximum(cnt_ref[...], 1.0)
    h = agg_ref[...] / cnt + jnp.dot(
        x_ref[...], wr_ref[0], preferred_element_type=_f32) + b_ref[0, 0]
    h = jnp.maximum(h, 0.0)
    h_ref[...] = h
    t_ref[...] = jnp.dot(h, wl_ref[0], preferred_element_type=_f32)


def _epi_trans(agg, cnt, xd, wrstack, bstack, wlstack):
    return pl.pallas_call(
        _epi_trans_body,
        grid=(NB,),
        in_specs=[
            pl.BlockSpec((BR, D), lambda i: (i, 0)),
            pl.BlockSpec((BR, 1), lambda i: (i, 0)),
            pl.BlockSpec((BR, D), lambda i: (i, 0)),
            pl.BlockSpec((1, D, D), lambda i: (i // HB, 0, 0)),
            pl.BlockSpec((1, 1, D), lambda i: (i // HB, 0, 0)),
            pl.BlockSpec((1, D, D), lambda i: (1 - i // HB, 0, 0)),
        ],
        out_specs=[
            pl.BlockSpec((BR, D), lambda i: (i, 0)),
            pl.BlockSpec((BR, D), lambda i: ((i + HB) % NB, 0)),
        ],
        out_shape=[jax.ShapeDtypeStruct((ND, D), _f32),
                   jax.ShapeDtypeStruct((ND, D), _f32)],
    )(agg, cnt[:, None], xd, wrstack, bstack[:, None], wlstack)


MR = 2000
MB = EL // MR
_bf16 = jnp.bfloat16


def _mlp_body(gu_ref, gi_ref, ea_ref, w1u_ref, w1i_ref, w1e_ref, b1_ref,
              w2_ref, b2_ref, w3_ref, b3_ref, o_ref):
    gu = gu_ref[...].astype(_bf16)
    gi = gi_ref[...].astype(_bf16)
    ea = ea_ref[...].astype(_bf16)
    z = (jnp.dot(gu, w1u_ref[...], preferred_element_type=_f32)
         + jnp.dot(gi, w1i_ref[...], preferred_element_type=_f32)
         + jnp.dot(ea, w1e_ref[...], preferred_element_type=_f32)
         + b1_ref[...])
    z = jnp.maximum(z, 0.0).astype(_bf16)
    z = jnp.maximum(jnp.dot(z, w2_ref[...], preferred_element_type=_f32)
                    + b2_ref[...], 0.0).astype(_bf16)
    o_ref[...] = jnp.dot(z, w3_ref[...], preferred_element_type=_f32) + b3_ref[...]


def _mlp(g, ea, w1u, w1i, w1e, b1, w2, b2, w3, b3):
    full = lambda i: (0, 0)
    return pl.pallas_call(
        _mlp_body,
        grid=(MB,),
        in_specs=[
            pl.BlockSpec((MR, D), lambda i: (i, 0)),
            pl.BlockSpec((MR, D), lambda i: (i + MB, 0)),
            pl.BlockSpec((MR, DE), lambda i: (i, 0)),
            pl.BlockSpec((D, D), full),
            pl.BlockSpec((D, D), full),
            pl.BlockSpec((DE, D), full),
            pl.BlockSpec((1, D), full),
            pl.BlockSpec((D, D), full),
            pl.BlockSpec((1, D), full),
            pl.BlockSpec((D, 2), full),
            pl.BlockSpec((1, 2), full),
        ],
        out_specs=pl.BlockSpec((MR, 2), lambda i: (i, 0)),
        out_shape=jax.ShapeDtypeStruct((EL, 2), _f32),
    )(g, g, ea, w1u, w1i, w1e, b1, w2, b2, w3, b3)


def kernel(x_user, x_item, edge_attr, Wl0_ui, bl0_ui, Wr0_ui, Wl0_iu, bl0_iu,
           Wr0_iu, Wl1_ui, bl1_ui, Wr1_ui, Wl1_iu, bl1_iu, Wr1_iu, W1, b1,
           W2, b2, W3, b3, ei_ui, ei_iu, edge_label_index):
    xd = jnp.concatenate([x_item, x_user], axis=0)
    npad_e = EPC - E
    pad_src = jnp.arange(npad_e, dtype=jnp.int32) % N
    pad_dst = N + (jnp.arange(npad_e, dtype=jnp.int32) % (NPAD - N))
    srcs2d = jnp.concatenate([
        ei_ui[0], pad_src, ei_iu[0] + N, pad_src,
    ]).reshape(2 * EPC // CH, CH)
    dsts2d = jnp.concatenate([
        ei_ui[1], pad_dst, ei_iu[1], pad_dst,
    ]).reshape(2 * EPC // CH, CH)

    t0 = _transform(xd, jnp.stack([Wl0_ui, Wl0_iu]))
    agg0, cnt = _sc_agg(t0, srcs2d, dsts2d, True)
    hd, t1 = _epi_trans(agg0, cnt, xd,
                        jnp.stack([Wr0_ui, Wr0_iu]),
                        jnp.stack([bl0_ui, bl0_iu]),
                        jnp.stack([Wl1_ui, Wl1_iu]))
    agg1 = _sc_agg(t1, srcs2d, dsts2d, False)
    zd = _epilogue(agg1, cnt, hd,
                   jnp.stack([Wr1_ui, Wr1_iu]),
                   jnp.stack([bl1_ui, bl1_iu]), relu=False)

    dec_idx = jnp.concatenate([
        edge_label_index[0] + N,
        edge_label_index[1],
        (jnp.arange(GPAD - 2 * EL, dtype=jnp.int32) % N),
    ]).reshape(GPAD // GCH, GCH)
    g = _sc_gather(zd, dec_idx)
    return _mlp(g, edge_attr,
                W1[0:D].astype(_bf16), W1[D:2 * D].astype(_bf16),
                W1[2 * D:].astype(_bf16), b1[None],
                W2.astype(_bf16), b2[None], W3.astype(_bf16), b3[None])
